# baseline (speedup 1.0000x reference)
import functools

import jax
import jax.numpy as jnp
from jax import lax
from jax.experimental import pallas as pl
from jax.experimental.pallas import tpu as pltpu
from jax.experimental.pallas import tpu_sc as plsc

DIM = 128
MAX_POSITIONS = 10000
BATCH = 16384

_INFO = plsc.get_sparse_core_info()
_NC, _NS, _L = _INFO.num_cores, _INFO.num_subcores, _INFO.num_lanes
_NW = _NC * _NS
_BPW = BATCH // _NW
_CHUNKS = 4
_CPW = _BPW // _CHUNKS


def _embed_kernel(x_hbm, table_hbm, out_hbm, x_v,
                  i0, i1, i2, i3, r0, r1, r2, r3,
                  g0, g1, g2, g3, wsem):
    wid = lax.axis_index("s") * _NC + lax.axis_index("c")
    base = wid * _BPW
    idxs = (i0, i1, i2, i3)
    bufs = (r0, r1, r2, r3)
    gsems = (g0, g1, g2, g3)

    pltpu.sync_copy(x_hbm.at[pl.ds(base, _BPW)], x_v)

    scale = jnp.float32(MAX_POSITIONS - 1)
    hi = jnp.int32(MAX_POSITIONS - 1)
    lo = jnp.int32(0)
    gathers = []
    for c in range(_CHUNKS):
        for i in range(_CPW // _L):
            xv = x_v[pl.ds(c * _CPW + i * _L, _L)]
            pos = (xv * scale).astype(jnp.int32)
            idxs[c][pl.ds(i * _L, _L)] = jnp.minimum(jnp.maximum(pos, lo), hi)
        gathers.append(pltpu.async_copy(table_hbm.at[idxs[c]], bufs[c], gsems[c]))

    writes = []
    for c in range(_CHUNKS):
        gathers[c].wait()
        writes.append(pltpu.async_copy(
            bufs[c], out_hbm.at[pl.ds(base + c * _CPW, _CPW)], wsem))
    for w in writes:
        w.wait()


@jax.jit
def kernel(x, embedding):
    mesh = plsc.VectorSubcoreMesh(core_axis_name="c", subcore_axis_name="s")
    run = functools.partial(
        pl.kernel,
        mesh=mesh,
        out_type=jax.ShapeDtypeStruct((BATCH, DIM), jnp.float32),
        scratch_types=(
            [pltpu.VMEM((_BPW,), jnp.float32)]
            + [pltpu.VMEM((_CPW,), jnp.int32) for _ in range(_CHUNKS)]
            + [pltpu.VMEM((_CPW, DIM), jnp.float32) for _ in range(_CHUNKS)]
            + [pltpu.SemaphoreType.DMA for _ in range(_CHUNKS + 1)]
        ),
    )(_embed_kernel)
    return run(x, embedding)


# speedup vs baseline: 1.0242x; 1.0242x over previous
import functools

import jax
import jax.numpy as jnp
from jax import lax
from jax.experimental import pallas as pl
from jax.experimental.pallas import tpu as pltpu
from jax.experimental.pallas import tpu_sc as plsc

DIM = 128
MAX_POSITIONS = 10000
BATCH = 16384

_INFO = plsc.get_sparse_core_info()
_NC, _NS, _L = _INFO.num_cores, _INFO.num_subcores, _INFO.num_lanes
_NW = _NC * _NS
_BPW = BATCH // _NW


def _embed_kernel(x_hbm, table_hbm, out_hbm, x_v, idx_v, rows_v, sem):
    wid = lax.axis_index("s") * _NC + lax.axis_index("c")
    base = wid * _BPW

    pltpu.sync_copy(x_hbm.at[pl.ds(base, _BPW)], x_v)

    scale = jnp.float32(MAX_POSITIONS - 1)
    hi = jnp.int32(MAX_POSITIONS - 1)
    lo = jnp.int32(0)

    def body(i, carry):
        off = i * _L
        xv = x_v[pl.ds(off, _L)]
        pos = (xv * scale).astype(jnp.int32)
        idx_v[pl.ds(off, _L)] = jnp.minimum(jnp.maximum(pos, lo), hi)
        return carry

    lax.fori_loop(0, _BPW // _L, body, 0)

    pltpu.async_copy(table_hbm.at[idx_v], rows_v, sem).wait()
    pltpu.sync_copy(rows_v, out_hbm.at[pl.ds(base, _BPW)])


@jax.jit
def kernel(x, embedding):
    mesh = plsc.VectorSubcoreMesh(core_axis_name="c", subcore_axis_name="s")
    run = functools.partial(
        pl.kernel,
        mesh=mesh,
        out_type=jax.ShapeDtypeStruct((BATCH, DIM), jnp.float32),
        scratch_types=[
            pltpu.VMEM((_BPW,), jnp.float32),
            pltpu.VMEM((_BPW,), jnp.int32),
            pltpu.VMEM((_BPW, DIM), jnp.float32),
            pltpu.SemaphoreType.DMA,
        ],
    )(_embed_kernel)
    return run(x, embedding)


# speedup vs baseline: 1.0280x; 1.0037x over previous
import functools

import jax
import jax.numpy as jnp
from jax import lax
from jax.experimental import pallas as pl
from jax.experimental.pallas import tpu as pltpu
from jax.experimental.pallas import tpu_sc as plsc

DIM = 128
MAX_POSITIONS = 10000
BATCH = 16384

_INFO = plsc.get_sparse_core_info()
_NC, _NS, _L = _INFO.num_cores, _INFO.num_subcores, _INFO.num_lanes
_NW = _NC * _NS
_BPW = BATCH // _NW


_HPW = _BPW // 2


def _embed_kernel(x_hbm, table_hbm, out_hbm, x_v, i0, i1, r0, r1,
                  g0, g1, wsem):
    wid = lax.axis_index("s") * _NC + lax.axis_index("c")
    base = wid * _BPW

    pltpu.sync_copy(x_hbm.at[pl.ds(base, _BPW)], x_v)

    scale = jnp.float32(MAX_POSITIONS - 1)
    hi = jnp.int32(MAX_POSITIONS - 1)
    lo = jnp.int32(0)

    def body(i, carry):
        off = i * _L
        xa = x_v[pl.ds(off, _L)]
        xb = x_v[pl.ds(_HPW + off, _L)]
        pa = (xa * scale).astype(jnp.int32)
        pb = (xb * scale).astype(jnp.int32)
        i0[pl.ds(off, _L)] = jnp.minimum(jnp.maximum(pa, lo), hi)
        i1[pl.ds(off, _L)] = jnp.minimum(jnp.maximum(pb, lo), hi)
        return carry

    lax.fori_loop(0, _HPW // _L, body, 0)

    ga = pltpu.async_copy(table_hbm.at[i0], r0, g0)
    gb = pltpu.async_copy(table_hbm.at[i1], r1, g1)
    ga.wait()
    wa = pltpu.async_copy(r0, out_hbm.at[pl.ds(base, _HPW)], wsem)
    gb.wait()
    wb = pltpu.async_copy(r1, out_hbm.at[pl.ds(base + _HPW, _HPW)], wsem)
    wa.wait()
    wb.wait()


@jax.jit
def kernel(x, embedding):
    mesh = plsc.VectorSubcoreMesh(core_axis_name="c", subcore_axis_name="s")
    run = functools.partial(
        pl.kernel,
        mesh=mesh,
        out_type=jax.ShapeDtypeStruct((BATCH, DIM), jnp.float32),
        scratch_types=[
            pltpu.VMEM((_BPW,), jnp.float32),
            pltpu.VMEM((_HPW,), jnp.int32),
            pltpu.VMEM((_HPW,), jnp.int32),
            pltpu.VMEM((_HPW, DIM), jnp.float32),
            pltpu.VMEM((_HPW, DIM), jnp.float32),
            pltpu.SemaphoreType.DMA,
            pltpu.SemaphoreType.DMA,
            pltpu.SemaphoreType.DMA,
        ],
    )(_embed_kernel)
    return run(x, embedding)
